# baseline (device time: 44014 ns/iter reference)
import jax
import jax.numpy as jnp
from jax import lax
from jax.experimental import pallas as pl
from jax.experimental.pallas import tpu as pltpu

N_DEV = 4
N_LAYERS = 3


def kernel(x, Win0, Wout0, Win1, Wout1, Win2, Wout2):
    m_per, d = x.shape
    h = Win0.shape[1]
    hh = h // 2

    def body(x_ref, win0_ref, wout0_ref, win1_ref, wout1_ref, win2_ref,
             wout2_ref, out_ref,
             xb, mywin, mywout, lwin, lwout, rwin, rwout,
             relAwin, relAwout, relBwin, relBwout,
             dwinA, dwoutA, dwinB, dwoutB,
             wssem, wrsem):
        j = lax.axis_index("i")
        left = lax.rem(j + N_DEV - 1, N_DEV)
        right = lax.rem(j + 1, N_DEV)

        barrier_sem = pltpu.get_barrier_semaphore()
        for nbr in (left, right):
            pl.semaphore_signal(barrier_sem, inc=1, device_id=(nbr,),
                                device_id_type=pl.DeviceIdType.MESH)
        pl.semaphore_wait(barrier_sem, 2)

        def fp(src_ref, win, wout):
            hact = jnp.maximum(
                jnp.dot(src_ref[...], win,
                        preferred_element_type=jnp.float32), 0.0)
            return jnp.dot(hact.astype(jnp.bfloat16), wout,
                           preferred_element_type=jnp.float32)

        def copy(src, dst, s, dev):
            return pltpu.make_async_remote_copy(
                src_ref=src, dst_ref=dst, send_sem=wssem.at[s],
                recv_sem=wrsem.at[s], device_id=(dev,),
                device_id_type=pl.DeviceIdType.MESH)

        win_refs = [win0_ref, win1_ref, win2_ref]
        wout_refs = [wout0_ref, wout1_ref, wout2_ref]

        direct = []
        all_rdmas = []
        for l in range(N_LAYERS):
            mywin[l] = win_refs[l][...].astype(jnp.bfloat16)
            mywout[l] = wout_refs[l][...].astype(jnp.bfloat16)
            sL1 = copy(mywin.at[l], rwin.at[l], 8 * l + 0, left)
            sL2 = copy(mywout.at[l], rwout.at[l], 8 * l + 1, left)
            sR1 = copy(mywin.at[l], lwin.at[l], 8 * l + 2, right)
            sR2 = copy(mywout.at[l], lwout.at[l], 8 * l + 3, right)
            for r in (sL1, sL2, sR1, sR2):
                r.start()
                all_rdmas.append(r)
            direct.append((sL1, sL2, sR1, sR2))

        xb[...] = x_ref[...].astype(jnp.bfloat16)

        for l in range(N_LAYERS):
            sL1, sL2, sR1, sR2 = direct[l]
            own = fp(xb, mywin[l], mywout[l])

            sR1.wait_recv()
            sR2.wait_recv()
            relAwin[l] = lwin[l, :, :hh]
            relAwout[l] = lwout[l, :hh, :]
            rA1 = copy(relAwin.at[l], dwinA.at[l], 8 * l + 4, right)
            rA2 = copy(relAwout.at[l], dwoutA.at[l], 8 * l + 5, right)
            rA1.start()
            rA2.start()
            all_rdmas.extend((rA1, rA2))

            sL1.wait_recv()
            sL2.wait_recv()
            relBwin[l] = rwin[l, :, hh:]
            relBwout[l] = rwout[l, hh:, :]
            rB1 = copy(relBwin.at[l], dwinB.at[l], 8 * l + 6, left)
            rB2 = copy(relBwout.at[l], dwoutB.at[l], 8 * l + 7, left)
            rB1.start()
            rB2.start()
            all_rdmas.extend((rB1, rB2))

            acc = (own + fp(xb, lwin[l], lwout[l])
                   + fp(xb, rwin[l], rwout[l]))

            rA1.wait_recv()
            rA2.wait_recv()
            rB1.wait_recv()
            rB2.wait_recv()
            res = (acc + fp(xb, dwinA[l], dwoutA[l])
                   + fp(xb, dwinB[l], dwoutB[l]))
            if l < N_LAYERS - 1:
                xb[...] = res.astype(jnp.bfloat16)
            else:
                out_ref[...] = res

        for r in all_rdmas:
            r.wait_send()

    winf = lambda: pltpu.VMEM((N_LAYERS, d, h), jnp.bfloat16)
    woutf = lambda: pltpu.VMEM((N_LAYERS, h, d), jnp.bfloat16)
    winh = lambda: pltpu.VMEM((N_LAYERS, d, hh), jnp.bfloat16)
    wouth = lambda: pltpu.VMEM((N_LAYERS, hh, d), jnp.bfloat16)
    return pl.pallas_call(
        body,
        out_shape=jax.ShapeDtypeStruct((m_per, d), jnp.float32),
        in_specs=[pl.BlockSpec(memory_space=pltpu.VMEM)] * 7,
        out_specs=pl.BlockSpec(memory_space=pltpu.VMEM),
        scratch_shapes=[
            pltpu.VMEM((m_per, d), jnp.bfloat16),
            winf(), woutf(),
            winf(), woutf(),
            winf(), woutf(),
            winh(), wouth(),
            winh(), wouth(),
            winh(), wouth(),
            winh(), wouth(),
            pltpu.SemaphoreType.DMA((24,)),
            pltpu.SemaphoreType.DMA((24,)),
        ],
        compiler_params=pltpu.CompilerParams(collective_id=0),
    )(x, Win0, Wout0, Win1, Wout1, Win2, Wout2)


# device time: 43884 ns/iter; 1.0030x vs baseline; 1.0030x over previous
import jax
import jax.numpy as jnp
from jax import lax
from jax.experimental import pallas as pl
from jax.experimental.pallas import tpu as pltpu

N_DEV = 4
N_LAYERS = 3


def kernel(x, Win0, Wout0, Win1, Wout1, Win2, Wout2):
    m_per, d = x.shape
    mh = m_per // 2

    def body(x_ref, win0_ref, wout0_ref, win1_ref, wout1_ref, win2_ref,
             wout2_ref, out_ref,
             xbT, xbB, agLT, agRT, agDT, agLB, agRB, agDB,
             pLT, pRT, pDT, pLB, pRB, pDB,
             rsLT, rsRT, rsDT, rsLB, rsRB, rsDB,
             winb, woutb, ssem, rsem):
        j = lax.axis_index("i")
        left = lax.rem(j + N_DEV - 1, N_DEV)
        right = lax.rem(j + 1, N_DEV)
        diag = lax.rem(j + 2, N_DEV)

        barrier_sem = pltpu.get_barrier_semaphore()
        for nbr in (left, right, diag):
            pl.semaphore_signal(barrier_sem, inc=1, device_id=(nbr,),
                                device_id_type=pl.DeviceIdType.MESH)
        pl.semaphore_wait(barrier_sem, 3)

        def fp(src_ref, l):
            hact = jnp.maximum(
                jnp.dot(src_ref[...], winb[l],
                        preferred_element_type=jnp.float32), 0.0)
            return jnp.dot(hact.astype(jnp.bfloat16), woutb[l],
                           preferred_element_type=jnp.float32)

        def copy(src, dst, s, dev):
            return pltpu.make_async_remote_copy(
                src_ref=src, dst_ref=dst, send_sem=ssem.at[s],
                recv_sem=rsem.at[s], device_id=(dev,),
                device_id_type=pl.DeviceIdType.MESH)

        T = dict(base=0, xb=xbT, agL=agLT, agR=agRT, agD=agDT,
                 pL=pLT, pR=pRT, pD=pDT, rsL=rsLT, rsR=rsRT, rsD=rsDT)
        B = dict(base=6, xb=xbB, agL=agLB, agR=agRB, agD=agDB,
                 pL=pLB, pR=pRB, pD=pDB, rsL=rsLB, rsR=rsRB, rsD=rsDB)

        def start_phase(S):
            b = S["base"]
            agl = copy(S["xb"], S["agL"], b + 0, right)
            agr = copy(S["xb"], S["agR"], b + 1, left)
            agd = copy(S["xb"], S["agD"], b + 2, diag)
            agd.start()
            agl.start()
            agr.start()
            S["ag"] = (agl, agr, agd)

        def finish_phase(S, l):
            b = S["base"]
            agl, agr, agd = S["ag"]
            own = fp(S["xb"], l)
            agl.wait()
            S["pL"][...] = fp(S["agL"], l).astype(jnp.bfloat16)
            rsl = copy(S["pL"], S["rsR"], b + 3, left)
            rsl.start()
            agr.wait()
            S["pR"][...] = fp(S["agR"], l).astype(jnp.bfloat16)
            rsr = copy(S["pR"], S["rsL"], b + 4, right)
            rsr.start()
            agd.wait()
            S["pD"][...] = fp(S["agD"], l).astype(jnp.bfloat16)
            rsd = copy(S["pD"], S["rsD"], b + 5, diag)
            rsd.start()
            rsl.wait()
            rsr.wait()
            rsd.wait()
            return (own + S["rsL"][...].astype(jnp.float32)
                    + S["rsR"][...].astype(jnp.float32)
                    + S["rsD"][...].astype(jnp.float32))

        xbT[...] = x_ref[:mh, :].astype(jnp.bfloat16)
        xbB[...] = x_ref[mh:, :].astype(jnp.bfloat16)
        start_phase(T)
        start_phase(B)
        win_refs = [win0_ref, win1_ref, win2_ref]
        wout_refs = [wout0_ref, wout1_ref, wout2_ref]
        for l in range(N_LAYERS):
            winb[l] = win_refs[l][...].astype(jnp.bfloat16)
            woutb[l] = wout_refs[l][...].astype(jnp.bfloat16)

        for l in range(N_LAYERS):
            resT = finish_phase(T, l)
            if l < N_LAYERS - 1:
                xbT[...] = resT.astype(jnp.bfloat16)
                start_phase(T)
            else:
                out_ref[:mh, :] = resT
            resB = finish_phase(B, l)
            if l < N_LAYERS - 1:
                xbB[...] = resB.astype(jnp.bfloat16)
                start_phase(B)
            else:
                out_ref[mh:, :] = resB

    bufb = lambda: pltpu.VMEM((mh, d), jnp.bfloat16)
    return pl.pallas_call(
        body,
        out_shape=jax.ShapeDtypeStruct((m_per, d), jnp.float32),
        in_specs=[pl.BlockSpec(memory_space=pltpu.VMEM)] * 7,
        out_specs=pl.BlockSpec(memory_space=pltpu.VMEM),
        scratch_shapes=[
            bufb(), bufb(),
            bufb(), bufb(), bufb(),
            bufb(), bufb(), bufb(),
            bufb(), bufb(), bufb(),
            bufb(), bufb(), bufb(),
            bufb(), bufb(), bufb(),
            bufb(), bufb(), bufb(),
            pltpu.VMEM((N_LAYERS,) + Win0.shape, jnp.bfloat16),
            pltpu.VMEM((N_LAYERS,) + Wout0.shape, jnp.bfloat16),
            pltpu.SemaphoreType.DMA((12,)),
            pltpu.SemaphoreType.DMA((12,)),
        ],
        compiler_params=pltpu.CompilerParams(collective_id=0),
    )(x, Win0, Wout0, Win1, Wout1, Win2, Wout2)


# device time: 42165 ns/iter; 1.0439x vs baseline; 1.0408x over previous
import jax
import jax.numpy as jnp
from jax import lax
from jax.experimental import pallas as pl
from jax.experimental.pallas import tpu as pltpu

N_DEV = 4


def kernel(x, Win0, Wout0, Win1, Wout1, Win2, Wout2):
    m_per, d = x.shape

    def body(x_ref, win0_ref, wout0_ref, win1_ref, wout1_ref, win2_ref,
             wout2_ref, out_ref,
             xcur, xb, agL, agR, agD, pj, pLb, pRb, pDb,
             rsFromL, rsFromR, rsFromD, winb, woutb, ssem, rsem):
        j = lax.axis_index("i")
        left = lax.rem(j + N_DEV - 1, N_DEV)
        right = lax.rem(j + 1, N_DEV)
        diag = lax.rem(j + 2, N_DEV)

        barrier_sem = pltpu.get_barrier_semaphore()
        for nbr in (left, right, diag):
            pl.semaphore_signal(barrier_sem, inc=1, device_id=(nbr,),
                                device_id_type=pl.DeviceIdType.MESH)
        pl.semaphore_wait(barrier_sem, 3)

        def mlp(src_ref):
            h = jnp.maximum(
                jnp.dot(src_ref[...], winb[...],
                        preferred_element_type=jnp.float32), 0.0)
            return jnp.dot(h.astype(jnp.bfloat16), woutb[...],
                           preferred_element_type=jnp.float32)

        def copy(src, dst, s, r, dev):
            return pltpu.make_async_remote_copy(
                src_ref=src, dst_ref=dst, send_sem=ssem.at[s],
                recv_sem=rsem.at[r], device_id=(dev,),
                device_id_type=pl.DeviceIdType.MESH)

        xcur[...] = x_ref[...]

        layers = [(win0_ref, wout0_ref), (win1_ref, wout1_ref),
                  (win2_ref, wout2_ref)]
        for l, (win_ref, wout_ref) in enumerate(layers):
            xb[...] = xcur[...].astype(jnp.bfloat16)
            agd = copy(xb, agD, 2, 2, diag)
            agl = copy(xb, agL, 0, 0, right)
            agr = copy(xb, agR, 1, 1, left)
            agd.start()
            agl.start()
            agr.start()
            winb[...] = win_ref[...].astype(jnp.bfloat16)
            woutb[...] = wout_ref[...].astype(jnp.bfloat16)
            pj[...] = mlp(xb)

            agl.wait()
            pLb[...] = mlp(agL).astype(jnp.bfloat16)
            rsl = copy(pLb, rsFromR, 3, 3, left)
            rsl.start()
            agr.wait()
            pRb[...] = mlp(agR).astype(jnp.bfloat16)
            rsr = copy(pRb, rsFromL, 4, 4, right)
            rsr.start()
            agd.wait()
            pDb[...] = mlp(agD).astype(jnp.bfloat16)
            rsd = copy(pDb, rsFromD, 5, 5, diag)
            rsd.start()
            rsl.wait()
            rsr.wait()
            rsd.wait()

            res = (pj[...] + rsFromL[...].astype(jnp.float32)
                   + rsFromR[...].astype(jnp.float32)
                   + rsFromD[...].astype(jnp.float32))
            if l < len(layers) - 1:
                xcur[...] = res
            else:
                out_ref[...] = res

    bufb = lambda: pltpu.VMEM((m_per, d), jnp.bfloat16)
    return pl.pallas_call(
        body,
        out_shape=jax.ShapeDtypeStruct((m_per, d), jnp.float32),
        in_specs=[pl.BlockSpec(memory_space=pltpu.VMEM)] * 7,
        out_specs=pl.BlockSpec(memory_space=pltpu.VMEM),
        scratch_shapes=[
            pltpu.VMEM((m_per, d), jnp.float32),
            bufb(),
            bufb(),
            bufb(),
            bufb(),
            pltpu.VMEM((m_per, d), jnp.float32),
            bufb(),
            bufb(),
            bufb(),
            bufb(),
            bufb(),
            bufb(),
            pltpu.VMEM(Win0.shape, jnp.bfloat16),
            pltpu.VMEM(Wout0.shape, jnp.bfloat16),
            pltpu.SemaphoreType.DMA((6,)),
            pltpu.SemaphoreType.DMA((6,)),
        ],
        compiler_params=pltpu.CompilerParams(collective_id=0),
    )(x, Win0, Wout0, Win1, Wout1, Win2, Wout2)


# device time: 38948 ns/iter; 1.1301x vs baseline; 1.0826x over previous
import jax
import jax.numpy as jnp
from jax import lax
from jax.experimental import pallas as pl
from jax.experimental.pallas import tpu as pltpu

N_DEV = 4


def kernel(x, Win0, Wout0, Win1, Wout1, Win2, Wout2):
    m_per, d = x.shape

    def body(x_ref, win0_ref, wout0_ref, win1_ref, wout1_ref, win2_ref,
             wout2_ref, out_ref,
             xcur, xb, agL, agR, agD, pj, pLb, pRb, pDb,
             rsFromL, rsFromR, rsFromD, winb, woutb, ssem, rsem):
        j = lax.axis_index("i")
        left = lax.rem(j + N_DEV - 1, N_DEV)
        right = lax.rem(j + 1, N_DEV)
        diag = lax.rem(j + 2, N_DEV)

        barrier_sem = pltpu.get_barrier_semaphore()
        for nbr in (left, right, diag):
            pl.semaphore_signal(barrier_sem, inc=1, device_id=(nbr,),
                                device_id_type=pl.DeviceIdType.MESH)
        pl.semaphore_wait(barrier_sem, 3)

        def mlp(src_ref):
            h = jnp.maximum(
                jnp.dot(src_ref[...], winb[...],
                        preferred_element_type=jnp.float32), 0.0)
            return jnp.dot(h.astype(jnp.bfloat16), woutb[...],
                           preferred_element_type=jnp.float32)

        def copy(src, dst, s, r, dev):
            return pltpu.make_async_remote_copy(
                src_ref=src, dst_ref=dst, send_sem=ssem.at[s],
                recv_sem=rsem.at[r], device_id=(dev,),
                device_id_type=pl.DeviceIdType.MESH)

        xcur[...] = x_ref[...]

        layers = [(win0_ref, wout0_ref), (win1_ref, wout1_ref),
                  (win2_ref, wout2_ref)]
        for l, (win_ref, wout_ref) in enumerate(layers):
            xb[...] = xcur[...].astype(jnp.bfloat16)
            agl = copy(xb, agL, 0, 0, right)
            agr = copy(xb, agR, 1, 1, left)
            agd = copy(xb, agD, 2, 2, diag)
            agl.start()
            agr.start()
            agd.start()
            winb[...] = win_ref[...].astype(jnp.bfloat16)
            woutb[...] = wout_ref[...].astype(jnp.bfloat16)
            pj[...] = mlp(xb)

            agl.wait()
            pLb[...] = mlp(agL).astype(jnp.bfloat16)
            rsl = copy(pLb, rsFromR, 3, 3, left)
            rsl.start()
            agr.wait()
            pRb[...] = mlp(agR).astype(jnp.bfloat16)
            rsr = copy(pRb, rsFromL, 4, 4, right)
            rsr.start()
            agd.wait()
            pDb[...] = mlp(agD).astype(jnp.bfloat16)
            rsd = copy(pDb, rsFromD, 5, 5, diag)
            rsd.start()
            rsl.wait()
            rsr.wait()
            rsd.wait()

            res = (pj[...] + rsFromL[...].astype(jnp.float32)
                   + rsFromR[...].astype(jnp.float32)
                   + rsFromD[...].astype(jnp.float32))
            if l < len(layers) - 1:
                xcur[...] = res
            else:
                out_ref[...] = res

    bufb = lambda: pltpu.VMEM((m_per, d), jnp.bfloat16)
    return pl.pallas_call(
        body,
        out_shape=jax.ShapeDtypeStruct((m_per, d), jnp.float32),
        in_specs=[pl.BlockSpec(memory_space=pltpu.VMEM)] * 7,
        out_specs=pl.BlockSpec(memory_space=pltpu.VMEM),
        scratch_shapes=[
            pltpu.VMEM((m_per, d), jnp.float32),
            bufb(),
            bufb(),
            bufb(),
            bufb(),
            pltpu.VMEM((m_per, d), jnp.float32),
            bufb(),
            bufb(),
            bufb(),
            bufb(),
            bufb(),
            bufb(),
            pltpu.VMEM(Win0.shape, jnp.bfloat16),
            pltpu.VMEM(Wout0.shape, jnp.bfloat16),
            pltpu.SemaphoreType.DMA((6,)),
            pltpu.SemaphoreType.DMA((6,)),
        ],
        compiler_params=pltpu.CompilerParams(collective_id=0),
    )(x, Win0, Wout0, Win1, Wout1, Win2, Wout2)


# device time: 38267 ns/iter; 1.1502x vs baseline; 1.0178x over previous
import jax
import jax.numpy as jnp
from jax import lax
from jax.experimental import pallas as pl
from jax.experimental.pallas import tpu as pltpu

N_DEV = 4
N_LAYERS = 3


def kernel(x, Win0, Wout0, Win1, Wout1, Win2, Wout2):
    m_per, d = x.shape
    h = Win0.shape[1]
    hh = h // 2

    def body(x_ref, win0_ref, wout0_ref, win1_ref, wout1_ref, win2_ref,
             wout2_ref, out_ref,
             xb, agL, agR, pLb, pRb, rsFromL, rsFromR,
             mywinA, mywoutA, mywinB, mywoutB,
             rwinA, rwoutA, lwinB, lwoutB,
             ssem, rsem, wssem, wrsem):
        j = lax.axis_index("i")
        left = lax.rem(j + N_DEV - 1, N_DEV)
        right = lax.rem(j + 1, N_DEV)

        barrier_sem = pltpu.get_barrier_semaphore()
        for nbr in (left, right):
            pl.semaphore_signal(barrier_sem, inc=1, device_id=(nbr,),
                                device_id_type=pl.DeviceIdType.MESH)
        pl.semaphore_wait(barrier_sem, 2)

        def fp(src_ref, win, wout):
            hact = jnp.maximum(
                jnp.dot(src_ref[...], win,
                        preferred_element_type=jnp.float32), 0.0)
            return jnp.dot(hact.astype(jnp.bfloat16), wout,
                           preferred_element_type=jnp.float32)

        def copy(src, dst, s_sem, r_sem, dev):
            return pltpu.make_async_remote_copy(
                src_ref=src, dst_ref=dst, send_sem=s_sem,
                recv_sem=r_sem, device_id=(dev,),
                device_id_type=pl.DeviceIdType.MESH)

        win_refs = [win0_ref, win1_ref, win2_ref]
        wout_refs = [wout0_ref, wout1_ref, wout2_ref]

        xb[...] = x_ref[...].astype(jnp.bfloat16)
        agl = copy(xb, agL, ssem.at[0], rsem.at[0], right)
        agr = copy(xb, agR, ssem.at[1], rsem.at[1], left)
        agl.start()
        agr.start()

        w_rdmas = []

        def ship(l):
            mywinA[l] = win_refs[l][:, :hh].astype(jnp.bfloat16)
            mywoutA[l] = wout_refs[l][:hh, :].astype(jnp.bfloat16)
            mywinB[l] = win_refs[l][:, hh:].astype(jnp.bfloat16)
            mywoutB[l] = wout_refs[l][hh:, :].astype(jnp.bfloat16)
            wa1 = copy(mywinA.at[l], rwinA.at[l], wssem.at[4 * l],
                       wrsem.at[4 * l], left)
            wa2 = copy(mywoutA.at[l], rwoutA.at[l], wssem.at[4 * l + 1],
                       wrsem.at[4 * l + 1], left)
            wb1 = copy(mywinB.at[l], lwinB.at[l], wssem.at[4 * l + 2],
                       wrsem.at[4 * l + 2], right)
            wb2 = copy(mywoutB.at[l], lwoutB.at[l], wssem.at[4 * l + 3],
                       wrsem.at[4 * l + 3], right)
            for r in (wa1, wa2, wb1, wb2):
                r.start()
            w_rdmas.append((wa1, wa2, wb1, wb2))

        ship(0)

        for l in range(N_LAYERS):
            if l > 0:
                agl = copy(xb, agL, ssem.at[0], rsem.at[0], right)
                agr = copy(xb, agR, ssem.at[1], rsem.at[1], left)
                agl.start()
                agr.start()
            wa1, wa2, wb1, wb2 = w_rdmas[l]
            own = (fp(xb, mywinA[l], mywoutA[l])
                   + fp(xb, mywinB[l], mywoutB[l]))

            agl.wait()
            sLo = fp(agL, mywinB[l], mywoutB[l])
            wa1.wait_recv()
            wa2.wait_recv()
            pLb[...] = (sLo + fp(agL, rwinA[l], rwoutA[l])
                        ).astype(jnp.bfloat16)
            rsl = copy(pLb, rsFromR, ssem.at[2], rsem.at[2], left)
            rsl.start()

            agr.wait()
            wb1.wait_recv()
            wb2.wait_recv()
            pRb[...] = (fp(agR, mywinA[l], mywoutA[l])
                        + fp(agR, lwinB[l], lwoutB[l])
                        ).astype(jnp.bfloat16)
            rsr = copy(pRb, rsFromL, ssem.at[3], rsem.at[3], right)
            rsr.start()
            if l + 1 < N_LAYERS:
                ship(l + 1)

            own2 = (fp(xb, rwinA[l], rwoutA[l])
                    + fp(xb, lwinB[l], lwoutB[l]))

            rsl.wait()
            rsr.wait()
            res = (own + own2 + rsFromL[...].astype(jnp.float32)
                   + rsFromR[...].astype(jnp.float32))
            if l < N_LAYERS - 1:
                xb[...] = res.astype(jnp.bfloat16)
            else:
                out_ref[...] = res

        for rds in w_rdmas:
            for r in rds:
                r.wait_send()

    bufb = lambda: pltpu.VMEM((m_per, d), jnp.bfloat16)
    winh = lambda: pltpu.VMEM((N_LAYERS, d, hh), jnp.bfloat16)
    wouth = lambda: pltpu.VMEM((N_LAYERS, hh, d), jnp.bfloat16)
    return pl.pallas_call(
        body,
        out_shape=jax.ShapeDtypeStruct((m_per, d), jnp.float32),
        in_specs=[pl.BlockSpec(memory_space=pltpu.VMEM)] * 7,
        out_specs=pl.BlockSpec(memory_space=pltpu.VMEM),
        scratch_shapes=[
            bufb(),
            bufb(),
            bufb(),
            bufb(),
            bufb(),
            bufb(),
            bufb(),
            winh(), wouth(),
            winh(), wouth(),
            winh(), wouth(),
            winh(), wouth(),
            pltpu.SemaphoreType.DMA((4,)),
            pltpu.SemaphoreType.DMA((4,)),
            pltpu.SemaphoreType.DMA((12,)),
            pltpu.SemaphoreType.DMA((12,)),
        ],
        compiler_params=pltpu.CompilerParams(collective_id=0),
    )(x, Win0, Wout0, Win1, Wout1, Win2, Wout2)
